# Initial kernel scaffold; baseline (speedup 1.0000x reference)
import jax
import jax.numpy as jnp
from jax.experimental import pallas as pl


def kernel(base_obs, emb_table, W, b):
    raise NotImplementedError("write your pallas kernel here")


# speedup vs baseline: 2.0745x; 2.0745x over previous
import functools

import jax
import jax.numpy as jnp
from jax import lax
from jax.experimental import pallas as pl
from jax.experimental.pallas import tpu as pltpu
from jax.experimental.pallas import tpu_sc as plsc

NC = 2
NS = 16
NW = NC * NS
C = 128


@functools.lru_cache(maxsize=None)
def _make_gather(vocab: int, hid: int, btot: int):
    per_w = btot // NW
    nchunk = per_w // C
    mesh = plsc.VectorSubcoreMesh(
        core_axis_name="c", subcore_axis_name="s", num_cores=NC, num_subcores=NS
    )

    @functools.partial(
        pl.kernel,
        mesh=mesh,
        compiler_params=pltpu.CompilerParams(use_tc_tiling_on_sc=False),
        out_type=jax.ShapeDtypeStruct((btot, hid), jnp.float32),
        scratch_types=[
            pltpu.VMEM((nchunk, C), jnp.int32),
            pltpu.VMEM((C, hid), jnp.float32),
            pltpu.SemaphoreType.DMA,
        ],
    )
    def gather_k(idx_hbm, table_hbm, out_hbm, idx_v, rows_v, sem):
        wid = lax.axis_index("s") * NC + lax.axis_index("c")
        base = wid * per_w
        pltpu.sync_copy(idx_hbm.at[wid], idx_v)

        @pl.loop(0, nchunk)
        def _chunk(j):
            pltpu.async_copy(table_hbm.at[idx_v.at[j]], rows_v, sem).wait()
            pltpu.sync_copy(rows_v, out_hbm.at[pl.ds(base + j * C, C)])

    return gather_k


@functools.lru_cache(maxsize=None)
def _make_head(btot: int, hid: int, ycls: int):
    blk = 2048

    def mm(x_ref, w_ref, b_ref, o_ref):
        o_ref[...] = (
            jnp.dot(x_ref[...], w_ref[...], preferred_element_type=jnp.float32)
            + b_ref[...]
        )

    return pl.pallas_call(
        mm,
        grid=(btot // blk,),
        in_specs=[
            pl.BlockSpec((blk, hid), lambda i: (i, 0)),
            pl.BlockSpec((hid, ycls), lambda i: (0, 0)),
            pl.BlockSpec((1, ycls), lambda i: (0, 0)),
        ],
        out_specs=pl.BlockSpec((blk, ycls), lambda i: (i, 0)),
        out_shape=jax.ShapeDtypeStruct((btot, ycls), jnp.float32),
    )


def kernel(base_obs, emb_table, W, b):
    B, L = base_obs.shape
    vocab, hid = emb_table.shape
    ycls = W.shape[1]
    btot = B * L

    idx = base_obs.astype(jnp.int32).reshape(NW, btot // NW // C, C)
    gathered = _make_gather(vocab, hid, btot)(idx, emb_table)
    out = _make_head(btot, hid, ycls)(gathered, W, b.reshape(1, ycls))
    return out.reshape(B, L, ycls)


# speedup vs baseline: 3.0277x; 1.4595x over previous
import functools

import jax
import jax.numpy as jnp
from jax import lax
from jax.experimental import pallas as pl
from jax.experimental.pallas import tpu as pltpu
from jax.experimental.pallas import tpu_sc as plsc

NC = 2
NS = 16
NW = NC * NS
C = 128


@functools.lru_cache(maxsize=None)
def _make_proj(vocab: int, hid: int, ycls: int):
    blk = 5000

    def mm(x_ref, w_ref, b_ref, o_ref):
        o_ref[...] = (
            jnp.dot(x_ref[...], w_ref[...], preferred_element_type=jnp.float32)
            + b_ref[...]
        )

    return pl.pallas_call(
        mm,
        grid=(vocab // blk,),
        in_specs=[
            pl.BlockSpec((blk, hid), lambda i: (i, 0)),
            pl.BlockSpec((hid, ycls), lambda i: (0, 0)),
            pl.BlockSpec((1, ycls), lambda i: (0, 0)),
        ],
        out_specs=pl.BlockSpec((blk, ycls), lambda i: (i, 0)),
        out_shape=jax.ShapeDtypeStruct((vocab, ycls), jnp.float32),
    )


@functools.lru_cache(maxsize=None)
def _make_gather(vocab: int, ycls: int, btot: int):
    per_w = btot // NW
    nchunk = per_w // C
    assert nchunk % 2 == 0
    mesh = plsc.VectorSubcoreMesh(
        core_axis_name="c", subcore_axis_name="s", num_cores=NC, num_subcores=NS
    )

    @functools.partial(
        pl.kernel,
        mesh=mesh,
        out_type=jax.ShapeDtypeStruct((btot, ycls), jnp.float32),
        scratch_types=[
            pltpu.VMEM((nchunk, C), jnp.int32),
            pltpu.VMEM((2, C, ycls), jnp.float32),
            pltpu.SemaphoreType.DMA,
            pltpu.SemaphoreType.DMA,
        ],
    )
    def gather_k(idx_hbm, table_hbm, out_hbm, idx_v, rows_v, sem0, sem1):
        wid = lax.axis_index("s") * NC + lax.axis_index("c")
        base = wid * per_w
        pltpu.sync_copy(idx_hbm.at[wid], idx_v)

        cp0 = pltpu.async_copy(table_hbm.at[idx_v.at[0]], rows_v.at[0], sem0)

        @pl.loop(0, nchunk, step=2)
        def _pair(j):
            cp1 = pltpu.async_copy(
                table_hbm.at[idx_v.at[j + 1]], rows_v.at[1], sem1
            )
            cp0.wait()
            pltpu.sync_copy(rows_v.at[0], out_hbm.at[pl.ds(base + j * C, C)])

            @pl.when(j + 2 < nchunk)
            def _():
                pltpu.async_copy(
                    table_hbm.at[idx_v.at[j + 2]], rows_v.at[0], sem0
                )

            cp1.wait()
            pltpu.sync_copy(
                rows_v.at[1], out_hbm.at[pl.ds(base + (j + 1) * C, C)]
            )

    return gather_k


def kernel(base_obs, emb_table, W, b):
    B, L = base_obs.shape
    vocab, hid = emb_table.shape
    ycls = W.shape[1]
    btot = B * L

    projT = _make_proj(vocab, hid, ycls)(emb_table, W, b.reshape(1, ycls))
    idx = base_obs.astype(jnp.int32).reshape(NW, btot // NW // C, C)
    out = _make_gather(vocab, ycls, btot)(idx, projT)
    return out.reshape(B, L, ycls)


# speedup vs baseline: 3.0360x; 1.0028x over previous
import functools

import jax
import jax.numpy as jnp
from jax import lax
from jax.experimental import pallas as pl
from jax.experimental.pallas import tpu as pltpu
from jax.experimental.pallas import tpu_sc as plsc

NC = 2
NS = 16
NW = NC * NS
C = 128


@functools.lru_cache(maxsize=None)
def _make_proj(vocab: int, hid: int, ycls: int):
    blk = 5000

    def mm(x_ref, w_ref, b_ref, o_ref):
        o_ref[...] = (
            jnp.dot(x_ref[...], w_ref[...], preferred_element_type=jnp.float32)
            + b_ref[...]
        )

    return pl.pallas_call(
        mm,
        grid=(vocab // blk,),
        in_specs=[
            pl.BlockSpec((blk, hid), lambda i: (i, 0)),
            pl.BlockSpec((hid, ycls), lambda i: (0, 0)),
            pl.BlockSpec((1, ycls), lambda i: (0, 0)),
        ],
        out_specs=pl.BlockSpec((blk, ycls), lambda i: (i, 0)),
        out_shape=jax.ShapeDtypeStruct((vocab, ycls), jnp.float32),
    )


@functools.lru_cache(maxsize=None)
def _make_gather(vocab: int, ycls: int, btot: int):
    per_w = btot // NW
    nchunk = per_w // C
    assert nchunk % 2 == 0
    mesh = plsc.VectorSubcoreMesh(
        core_axis_name="c", subcore_axis_name="s", num_cores=NC, num_subcores=NS
    )

    @functools.partial(
        pl.kernel,
        mesh=mesh,
        compiler_params=pltpu.CompilerParams(use_tc_tiling_on_sc=True),
        out_type=jax.ShapeDtypeStruct((btot, ycls), jnp.float32),
        scratch_types=[
            pltpu.VMEM((nchunk, C), jnp.int32),
            pltpu.VMEM((2, C, ycls), jnp.float32),
            pltpu.SemaphoreType.DMA,
            pltpu.SemaphoreType.DMA,
        ],
    )
    def gather_k(idx_hbm, table_hbm, out_hbm, idx_v, rows_v, sem0, sem1):
        wid = lax.axis_index("s") * NC + lax.axis_index("c")
        base = wid * per_w
        pltpu.sync_copy(idx_hbm.at[wid], idx_v)

        cp0 = pltpu.async_copy(table_hbm.at[idx_v.at[0]], rows_v.at[0], sem0)

        @pl.loop(0, nchunk, step=2)
        def _pair(j):
            cp1 = pltpu.async_copy(
                table_hbm.at[idx_v.at[j + 1]], rows_v.at[1], sem1
            )
            cp0.wait()
            pltpu.sync_copy(rows_v.at[0], out_hbm.at[pl.ds(base + j * C, C)])

            @pl.when(j + 2 < nchunk)
            def _():
                pltpu.async_copy(
                    table_hbm.at[idx_v.at[j + 2]], rows_v.at[0], sem0
                )

            cp1.wait()
            pltpu.sync_copy(
                rows_v.at[1], out_hbm.at[pl.ds(base + (j + 1) * C, C)]
            )

    return gather_k


def kernel(base_obs, emb_table, W, b):
    B, L = base_obs.shape
    vocab, hid = emb_table.shape
    ycls = W.shape[1]
    btot = B * L

    projT = _make_proj(vocab, hid, ycls)(emb_table, W, b.reshape(1, ycls))
    idx = base_obs.astype(jnp.int32).reshape(NW, btot // NW // C, C)
    out = _make_gather(vocab, ycls, btot)(idx, projT)
    return out.reshape(B, L, ycls)


# speedup vs baseline: 4.2086x; 1.3862x over previous
import functools

import jax
import jax.numpy as jnp
from jax import lax
from jax.experimental import pallas as pl
from jax.experimental.pallas import tpu as pltpu
from jax.experimental.pallas import tpu_sc as plsc

NC = 2
NS = 16
NW = NC * NS


@functools.lru_cache(maxsize=None)
def _make_proj(vocab: int, hid: int, ycls: int):
    blk = 5000

    def mm(x_ref, w_ref, b_ref, o_ref):
        o_ref[...] = (
            jnp.dot(x_ref[...], w_ref[...], preferred_element_type=jnp.float32)
            + b_ref[...]
        )

    return pl.pallas_call(
        mm,
        grid=(vocab // blk,),
        in_specs=[
            pl.BlockSpec((blk, hid), lambda i: (i, 0)),
            pl.BlockSpec((hid, ycls), lambda i: (0, 0)),
            pl.BlockSpec((1, ycls), lambda i: (0, 0)),
        ],
        out_specs=pl.BlockSpec((blk, ycls), lambda i: (i, 0)),
        out_shape=jax.ShapeDtypeStruct((vocab, ycls), jnp.float32),
    )


@functools.lru_cache(maxsize=None)
def _make_gather(vocab: int, ycls: int, B: int, L: int):
    per_w = B // NW
    assert per_w % 2 == 0
    mesh = plsc.VectorSubcoreMesh(
        core_axis_name="c", subcore_axis_name="s", num_cores=NC, num_subcores=NS
    )

    @functools.partial(
        pl.kernel,
        mesh=mesh,
        compiler_params=pltpu.CompilerParams(use_tc_tiling_on_sc=True),
        out_type=jax.ShapeDtypeStruct((B, L, ycls), jnp.float32),
        scratch_types=[
            pltpu.VMEM((per_w, L), jnp.int32),
            pltpu.VMEM((2, L, ycls), jnp.float32),
            pltpu.SemaphoreType.DMA,
            pltpu.SemaphoreType.DMA,
        ],
    )
    def gather_k(idx_hbm, table_hbm, out_hbm, idx_v, rows_v, sem0, sem1):
        wid = lax.axis_index("s") * NC + lax.axis_index("c")
        base = wid * per_w
        pltpu.sync_copy(idx_hbm.at[pl.ds(base, per_w)], idx_v)

        cp0 = pltpu.async_copy(table_hbm.at[idx_v.at[0]], rows_v.at[0], sem0)

        @pl.loop(0, per_w, step=2)
        def _pair(e):
            cp1 = pltpu.async_copy(
                table_hbm.at[idx_v.at[e + 1]], rows_v.at[1], sem1
            )
            cp0.wait()
            pltpu.sync_copy(rows_v.at[0], out_hbm.at[base + e])

            @pl.when(e + 2 < per_w)
            def _():
                pltpu.async_copy(
                    table_hbm.at[idx_v.at[e + 2]], rows_v.at[0], sem0
                )

            cp1.wait()
            pltpu.sync_copy(rows_v.at[1], out_hbm.at[base + e + 1])

    return gather_k


def kernel(base_obs, emb_table, W, b):
    B, L = base_obs.shape
    vocab, hid = emb_table.shape
    ycls = W.shape[1]

    projT = _make_proj(vocab, hid, ycls)(emb_table, W, b.reshape(1, ycls))
    out = _make_gather(vocab, ycls, B, L)(base_obs.astype(jnp.int32), projT)
    return out


# speedup vs baseline: 8.8570x; 2.1045x over previous
import functools

import jax
import jax.numpy as jnp
from jax import lax
from jax.experimental import pallas as pl
from jax.experimental.pallas import tpu as pltpu
from jax.experimental.pallas import tpu_sc as plsc

NC = 2
NS = 16
NW = NC * NS
C = 128


@functools.lru_cache(maxsize=None)
def _make_proj(vocab: int, hid: int, ycls: int):
    blk = 6400
    grid = (vocab + blk - 1) // blk

    def mm(xt_ref, w_ref, b_ref, o_ref):
        o_ref[...] = (
            lax.dot_general(
                xt_ref[...], w_ref[...],
                (((0,), (0,)), ((), ())),
                preferred_element_type=jnp.float32,
            )
            + b_ref[...]
        )

    return pl.pallas_call(
        mm,
        grid=(grid,),
        in_specs=[
            pl.BlockSpec((hid, blk), lambda i: (0, i)),
            pl.BlockSpec((hid, ycls), lambda i: (0, 0)),
            pl.BlockSpec((1, ycls), lambda i: (0, 0)),
        ],
        out_specs=pl.BlockSpec((blk, ycls), lambda i: (i, 0)),
        out_shape=jax.ShapeDtypeStruct((vocab, ycls), jnp.float32),
    )


@functools.lru_cache(maxsize=None)
def _make_gather(vocab: int, ycls: int, btot: int):
    per_w = btot // NW
    nchunk = per_w // C
    assert nchunk % 2 == 0
    mesh = plsc.VectorSubcoreMesh(
        core_axis_name="c", subcore_axis_name="s", num_cores=NC, num_subcores=NS
    )

    @functools.partial(
        pl.kernel,
        mesh=mesh,
        compiler_params=pltpu.CompilerParams(use_tc_tiling_on_sc=True),
        out_type=jax.ShapeDtypeStruct((btot, ycls), jnp.float32),
        scratch_types=[
            pltpu.VMEM((nchunk, C), jnp.int32),
            pltpu.VMEM((2, C, ycls), jnp.float32),
            pltpu.SemaphoreType.DMA,
            pltpu.SemaphoreType.DMA,
        ],
    )
    def gather_k(idx_hbm, table_hbm, out_hbm, idx_v, rows_v, sem0, sem1):
        wid = lax.axis_index("s") * NC + lax.axis_index("c")
        base = wid * per_w
        pltpu.sync_copy(idx_hbm.at[wid], idx_v)

        cp0 = pltpu.async_copy(table_hbm.at[idx_v.at[0]], rows_v.at[0], sem0)

        @pl.loop(0, nchunk, step=2)
        def _pair(j):
            cp1 = pltpu.async_copy(
                table_hbm.at[idx_v.at[j + 1]], rows_v.at[1], sem1
            )
            cp0.wait()
            pltpu.sync_copy(rows_v.at[0], out_hbm.at[pl.ds(base + j * C, C)])

            @pl.when(j + 2 < nchunk)
            def _():
                pltpu.async_copy(
                    table_hbm.at[idx_v.at[j + 2]], rows_v.at[0], sem0
                )

            cp1.wait()
            pltpu.sync_copy(
                rows_v.at[1], out_hbm.at[pl.ds(base + (j + 1) * C, C)]
            )

    return gather_k


def kernel(base_obs, emb_table, W, b):
    B, L = base_obs.shape
    vocab, hid = emb_table.shape
    ycls = W.shape[1]
    btot = B * L

    projT = _make_proj(vocab, hid, ycls)(emb_table.T, W, b.reshape(1, ycls))
    idx = base_obs.astype(jnp.int32).T.reshape(NW, btot // NW // C, C)
    out = _make_gather(vocab, ycls, btot)(idx, projT)
    return out.reshape(L, B, ycls).transpose(1, 0, 2)


# speedup vs baseline: 8.9161x; 1.0067x over previous
import functools

import jax
import jax.numpy as jnp
from jax import lax
from jax.experimental import pallas as pl
from jax.experimental.pallas import tpu as pltpu
from jax.experimental.pallas import tpu_sc as plsc

NC = 2
NS = 16
NW = NC * NS
C = 128


@functools.lru_cache(maxsize=None)
def _make_proj(vocab: int, hid: int, ycls: int):
    blk = 6400
    grid = (vocab + blk - 1) // blk

    def mm(xt_ref, w_ref, b_ref, o_ref):
        o_ref[...] = (
            lax.dot_general(
                xt_ref[...], w_ref[...],
                (((0,), (0,)), ((), ())),
                preferred_element_type=jnp.float32,
            )
            + b_ref[...]
        )

    return pl.pallas_call(
        mm,
        grid=(grid,),
        in_specs=[
            pl.BlockSpec((hid, blk), lambda i: (0, i)),
            pl.BlockSpec((hid, ycls), lambda i: (0, 0)),
            pl.BlockSpec((1, ycls), lambda i: (0, 0)),
        ],
        out_specs=pl.BlockSpec((blk, ycls), lambda i: (i, 0)),
        out_shape=jax.ShapeDtypeStruct((vocab, ycls), jnp.float32),
    )


@functools.lru_cache(maxsize=None)
def _make_gather(vocab: int, ycls: int, btot: int):
    per_w = btot // NW
    nchunk = per_w // C
    R = 4
    assert nchunk % R == 2
    mesh = plsc.VectorSubcoreMesh(
        core_axis_name="c", subcore_axis_name="s", num_cores=NC, num_subcores=NS
    )

    @functools.partial(
        pl.kernel,
        mesh=mesh,
        compiler_params=pltpu.CompilerParams(use_tc_tiling_on_sc=True),
        out_type=jax.ShapeDtypeStruct((btot, ycls), jnp.float32),
        scratch_types=[
            pltpu.VMEM((nchunk, C), jnp.int32),
            pltpu.VMEM((R, C, ycls), jnp.float32),
            [pltpu.SemaphoreType.DMA] * R,
            [pltpu.SemaphoreType.DMA] * R,
        ],
    )
    def gather_k(idx_hbm, table_hbm, out_hbm, idx_v, rows_v, gsems, ssems):
        wid = lax.axis_index("s") * NC + lax.axis_index("c")
        base = wid * per_w
        pltpu.sync_copy(idx_hbm.at[wid], idx_v)

        gd = [
            pltpu.async_copy(table_hbm.at[idx_v.at[t]], rows_v.at[t], gsems[t])
            for t in range(R)
        ]

        def store_wait(t):
            pltpu.make_async_copy(
                rows_v.at[t], out_hbm.at[pl.ds(base, C)], ssems[t]
            ).wait()

        @pl.loop(0, nchunk - 2, step=R)
        def _grp(j0):
            for t in range(R):
                j = j0 + t
                gd[t].wait()
                pltpu.async_copy(
                    rows_v.at[t], out_hbm.at[pl.ds(base + j * C, C)], ssems[t]
                )
                tp = (t + 2) % R

                @pl.when(jnp.logical_and(j >= 2, j + 2 < nchunk))
                def _():
                    store_wait(tp)
                    pltpu.async_copy(
                        table_hbm.at[idx_v.at[j + 2]], rows_v.at[tp], gsems[tp]
                    )

        for t in range(nchunk % R):
            j = nchunk - (nchunk % R) + t
            gd[t].wait()
            pltpu.async_copy(
                rows_v.at[t], out_hbm.at[pl.ds(base + j * C, C)], ssems[t]
            )
        for t in range(R):
            store_wait(t)

    return gather_k


def kernel(base_obs, emb_table, W, b):
    B, L = base_obs.shape
    vocab, hid = emb_table.shape
    ycls = W.shape[1]
    btot = B * L

    projT = _make_proj(vocab, hid, ycls)(emb_table.T, W, b.reshape(1, ycls))
    idx = base_obs.astype(jnp.int32).T.reshape(NW, btot // NW // C, C)
    out = _make_gather(vocab, ycls, btot)(idx, projT)
    return out.reshape(L, B, ycls).transpose(1, 0, 2)


# speedup vs baseline: 9.1847x; 1.0301x over previous
import functools

import jax
import jax.numpy as jnp
from jax import lax
from jax.experimental import pallas as pl
from jax.experimental.pallas import tpu as pltpu
from jax.experimental.pallas import tpu_sc as plsc

NC = 2
NS = 16
NW = NC * NS
C = 128


@functools.lru_cache(maxsize=None)
def _make_proj(vocab: int, hid: int, ycls: int):
    blk = 12800
    grid = (vocab + blk - 1) // blk

    def mm(xt_ref, w_ref, b_ref, o_ref):
        o_ref[...] = (
            lax.dot_general(
                xt_ref[...], w_ref[...],
                (((0,), (0,)), ((), ())),
                preferred_element_type=jnp.float32,
            )
            + b_ref[...]
        )

    return pl.pallas_call(
        mm,
        grid=(grid,),
        in_specs=[
            pl.BlockSpec((hid, blk), lambda i: (0, i)),
            pl.BlockSpec((hid, ycls), lambda i: (0, 0)),
            pl.BlockSpec((1, ycls), lambda i: (0, 0)),
        ],
        out_specs=pl.BlockSpec((blk, ycls), lambda i: (i, 0)),
        out_shape=jax.ShapeDtypeStruct((vocab, ycls), jnp.float32),
    )


@functools.lru_cache(maxsize=None)
def _make_gather(vocab: int, ycls: int, btot: int):
    per_w = btot // NW
    nchunk = per_w // C
    R = 4
    assert nchunk % R == 2
    mesh = plsc.VectorSubcoreMesh(
        core_axis_name="c", subcore_axis_name="s", num_cores=NC, num_subcores=NS
    )

    @functools.partial(
        pl.kernel,
        mesh=mesh,
        compiler_params=pltpu.CompilerParams(use_tc_tiling_on_sc=True),
        out_type=jax.ShapeDtypeStruct((btot, ycls), jnp.float32),
        scratch_types=[
            pltpu.VMEM((nchunk, C), jnp.int32),
            pltpu.VMEM((R, C, ycls), jnp.float32),
            [pltpu.SemaphoreType.DMA] * R,
            [pltpu.SemaphoreType.DMA] * R,
        ],
    )
    def gather_k(idx_hbm, table_hbm, out_hbm, idx_v, rows_v, gsems, ssems):
        wid = lax.axis_index("s") * NC + lax.axis_index("c")
        base = wid * per_w
        pltpu.sync_copy(idx_hbm.at[wid], idx_v)

        gd = [
            pltpu.async_copy(table_hbm.at[idx_v.at[t]], rows_v.at[t], gsems[t])
            for t in range(R)
        ]

        def store_wait(t):
            pltpu.make_async_copy(
                rows_v.at[t], out_hbm.at[pl.ds(base, C)], ssems[t]
            ).wait()

        @pl.loop(0, nchunk - 2, step=R)
        def _grp(j0):
            for t in range(R):
                j = j0 + t
                gd[t].wait()
                pltpu.async_copy(
                    rows_v.at[t], out_hbm.at[pl.ds(base + j * C, C)], ssems[t]
                )
                tp = (t + 2) % R

                @pl.when(jnp.logical_and(j >= 2, j + 2 < nchunk))
                def _():
                    store_wait(tp)
                    pltpu.async_copy(
                        table_hbm.at[idx_v.at[j + 2]], rows_v.at[tp], gsems[tp]
                    )

        for t in range(nchunk % R):
            j = nchunk - (nchunk % R) + t
            gd[t].wait()
            pltpu.async_copy(
                rows_v.at[t], out_hbm.at[pl.ds(base + j * C, C)], ssems[t]
            )
        for t in range(R):
            store_wait(t)

    return gather_k


def kernel(base_obs, emb_table, W, b):
    B, L = base_obs.shape
    vocab, hid = emb_table.shape
    ycls = W.shape[1]
    btot = B * L

    projT = _make_proj(vocab, hid, ycls)(emb_table.T, W, b.reshape(1, ycls))
    idx = base_obs.astype(jnp.int32).T.reshape(NW, btot // NW // C, C)
    out = _make_gather(vocab, ycls, btot)(idx, projT)
    return out.reshape(L, B, ycls).transpose(1, 0, 2)


# speedup vs baseline: 9.2590x; 1.0081x over previous
import functools

import jax
import jax.numpy as jnp
from jax import lax
from jax.experimental import pallas as pl
from jax.experimental.pallas import tpu as pltpu
from jax.experimental.pallas import tpu_sc as plsc

NC = 2
NS = 16
NW = NC * NS
C = 128


@functools.lru_cache(maxsize=None)
def _make_proj(vocab: int, hid: int, ycls: int):
    blk = 25600
    grid = (vocab + blk - 1) // blk

    def mm(xt_ref, w_ref, b_ref, o_ref):
        o_ref[...] = (
            lax.dot_general(
                xt_ref[...], w_ref[...],
                (((0,), (0,)), ((), ())),
                preferred_element_type=jnp.float32,
            )
            + b_ref[...]
        )

    return pl.pallas_call(
        mm,
        grid=(grid,),
        in_specs=[
            pl.BlockSpec((hid, blk), lambda i: (0, i)),
            pl.BlockSpec((hid, ycls), lambda i: (0, 0)),
            pl.BlockSpec((1, ycls), lambda i: (0, 0)),
        ],
        out_specs=pl.BlockSpec((blk, ycls), lambda i: (i, 0)),
        out_shape=jax.ShapeDtypeStruct((vocab, ycls), jnp.float32),
    )


@functools.lru_cache(maxsize=None)
def _make_gather(vocab: int, ycls: int, btot: int):
    per_w = btot // NW
    nchunk = per_w // C
    R = 4
    assert nchunk % R == 2
    mesh = plsc.VectorSubcoreMesh(
        core_axis_name="c", subcore_axis_name="s", num_cores=NC, num_subcores=NS
    )

    @functools.partial(
        pl.kernel,
        mesh=mesh,
        compiler_params=pltpu.CompilerParams(use_tc_tiling_on_sc=True),
        out_type=jax.ShapeDtypeStruct((btot, ycls), jnp.float32),
        scratch_types=[
            pltpu.VMEM((nchunk, C), jnp.int32),
            pltpu.VMEM((R, C, ycls), jnp.float32),
            [pltpu.SemaphoreType.DMA] * R,
            [pltpu.SemaphoreType.DMA] * R,
        ],
    )
    def gather_k(idx_hbm, table_hbm, out_hbm, idx_v, rows_v, gsems, ssems):
        wid = lax.axis_index("s") * NC + lax.axis_index("c")
        base = wid * per_w
        pltpu.sync_copy(idx_hbm.at[wid], idx_v)

        gd = [
            pltpu.async_copy(table_hbm.at[idx_v.at[t]], rows_v.at[t], gsems[t])
            for t in range(R)
        ]

        def store_wait(t):
            pltpu.make_async_copy(
                rows_v.at[t], out_hbm.at[pl.ds(base, C)], ssems[t]
            ).wait()

        @pl.loop(0, nchunk - 2, step=R)
        def _grp(j0):
            for t in range(R):
                j = j0 + t
                gd[t].wait()
                pltpu.async_copy(
                    rows_v.at[t], out_hbm.at[pl.ds(base + j * C, C)], ssems[t]
                )
                tp = (t + 2) % R

                @pl.when(jnp.logical_and(j >= 2, j + 2 < nchunk))
                def _():
                    store_wait(tp)
                    pltpu.async_copy(
                        table_hbm.at[idx_v.at[j + 2]], rows_v.at[tp], gsems[tp]
                    )

        for t in range(nchunk % R):
            j = nchunk - (nchunk % R) + t
            gd[t].wait()
            pltpu.async_copy(
                rows_v.at[t], out_hbm.at[pl.ds(base + j * C, C)], ssems[t]
            )
        for t in range(R):
            store_wait(t)

    return gather_k


def kernel(base_obs, emb_table, W, b):
    B, L = base_obs.shape
    vocab, hid = emb_table.shape
    ycls = W.shape[1]
    btot = B * L

    projT = _make_proj(vocab, hid, ycls)(emb_table.T, W, b.reshape(1, ycls))
    idx = base_obs.astype(jnp.int32).T.reshape(NW, btot // NW // C, C)
    out = _make_gather(vocab, ycls, btot)(idx, projT)
    return out.reshape(L, B, ycls).transpose(1, 0, 2)
